# Initial kernel scaffold; baseline (speedup 1.0000x reference)
import jax
import jax.numpy as jnp
from jax.experimental import pallas as pl


def kernel(node_attrs, node_feats, edge_attrs, edge_feats, edge_index, W1, Wm1, Wm2, W2, Wskip):
    raise NotImplementedError("write your pallas kernel here")


# speedup vs baseline: 1.4576x; 1.4576x over previous
import functools
import math

import jax
import jax.numpy as jnp
from jax import lax
from jax.experimental import pallas as pl
from jax.experimental.pallas import tpu as pltpu
from jax.experimental.pallas import tpu_sc as plsc

N = 10000
E = 320000
D_ATTR = 16
D_FEAT = 128
D_EFEAT = 8

NC = 2
NS = 16
NW = NC * NS

C = 128
K = C // 128
T = 10240
E_PAD = T * NW
CHUNKS = T // C
N_PAD = 10240
ROWS_PER_TILE = N_PAD // NS

_INV_SQRT_F = float(1.0 / math.sqrt(D_FEAT))
_INV_SQRT_E = float(1.0 / math.sqrt(D_EFEAT))
_INV_SQRT_SKIP = float(1.0 / math.sqrt(D_FEAT * D_ATTR))
_LOG2 = float(math.log(2.0))


def _h_body(nf_ref, w1_ref, out_ref):
    out_ref[...] = jnp.dot(nf_ref[...], w1_ref[...],
                           preferred_element_type=jnp.float32) * _INV_SQRT_F


def _compute_h(node_feats, W1):
    return pl.pallas_call(
        _h_body,
        out_shape=jax.ShapeDtypeStruct((N, D_FEAT), jnp.float32),
    )(node_feats, W1)


_EBLK = 4096


def _tpw_body(ef_ref, ea_ref, wm1_ref, wm2_ref, out_ref):
    t = jax.nn.softplus(jnp.dot(ef_ref[...], wm1_ref[...],
                                preferred_element_type=jnp.float32)
                        * _INV_SQRT_E) - _LOG2
    t = t * ea_ref[...] * _INV_SQRT_E
    out_ref[...] = jnp.dot(t, wm2_ref[...], preferred_element_type=jnp.float32)


def _compute_tpw(ef_pad, ea_pad, Wm1, Wm2):
    grid = (E_PAD // _EBLK,)
    return pl.pallas_call(
        _tpw_body,
        grid=grid,
        in_specs=[
            pl.BlockSpec((_EBLK, D_EFEAT), lambda i: (i, 0)),
            pl.BlockSpec((_EBLK, 1), lambda i: (i, 0)),
            pl.BlockSpec((D_EFEAT, D_EFEAT), lambda i: (0, 0)),
            pl.BlockSpec((D_EFEAT, D_FEAT), lambda i: (0, 0)),
        ],
        out_specs=pl.BlockSpec((_EBLK, D_FEAT), lambda i: (i, 0)),
        out_shape=jax.ShapeDtypeStruct((E_PAD, D_FEAT), jnp.float32),
    )(ef_pad, ea_pad, Wm1, Wm2)


def _sc_body(h_hbm, tpw_hbm, sidx_hbm, ridx_hbm, zeros_hbm, out_hbm,
             sidx_v, ridx_v, rows_v, tpw_v, m_shared, sem):
    cid = lax.axis_index("c")
    sid = lax.axis_index("s")
    wid = sid * NC + cid

    pltpu.sync_copy(zeros_hbm, m_shared.at[pl.ds(sid * ROWS_PER_TILE,
                                                 ROWS_PER_TILE)])
    plsc.subcore_barrier()

    def chunk(j, carry):
        rbase = wid * (T // 128) + j * K
        base = rbase * 128
        pltpu.sync_copy(sidx_hbm.at[pl.ds(rbase, K)], sidx_v)
        pltpu.sync_copy(ridx_hbm.at[pl.ds(rbase, K)], ridx_v)
        pltpu.sync_copy(tpw_hbm.at[pl.ds(base, C)], tpw_v)
        cps = [pltpu.async_copy(h_hbm.at[sidx_v.at[k]],
                                rows_v.at[pl.ds(k * 128, 128)], sem)
               for k in range(K)]
        for cp in cps:
            cp.wait()

        def edge(i, c):
            for cg in range(D_FEAT // 16):
                sl = pl.ds(cg * 16, 16)
                rows_v[i, sl] = rows_v[i, sl] * tpw_v[i, sl]
            return c

        lax.fori_loop(0, C, edge, 0, unroll=False)

        for k in range(K):
            pltpu.sync_copy(rows_v.at[pl.ds(k * 128, 128)],
                            m_shared.at[ridx_v.at[k]], add=True)
        return carry

    lax.fori_loop(0, CHUNKS, chunk, 0, unroll=False)
    plsc.subcore_barrier()

    pltpu.sync_copy(m_shared.at[pl.ds(sid * ROWS_PER_TILE, ROWS_PER_TILE)],
                    out_hbm.at[cid, pl.ds(sid * ROWS_PER_TILE, ROWS_PER_TILE)])


def _sc_scatter(h, tpw_pad, sidx2d, ridx2d, zeros_slab):
    mesh = plsc.VectorSubcoreMesh(core_axis_name="c", subcore_axis_name="s")
    fn = functools.partial(
        pl.kernel,
        out_type=jax.ShapeDtypeStruct((NC, N_PAD, D_FEAT), jnp.float32),
        mesh=mesh,
        scratch_types=[
            pltpu.VMEM((K, 128), jnp.int32),
            pltpu.VMEM((K, 128), jnp.int32),
            pltpu.VMEM((C, D_FEAT), jnp.float32),
            pltpu.VMEM((C, D_FEAT), jnp.float32),
            pltpu.VMEM_SHARED((N_PAD, D_FEAT), jnp.float32),
            pltpu.SemaphoreType.DMA,
        ],
    )(_sc_body)
    return fn(h, tpw_pad, sidx2d, ridx2d, zeros_slab)


_NBLK = 1000


def _final_body(mp_ref, attrs_ref, w2_ref, wskipT_ref, out_ref):
    m = (mp_ref[0] + mp_ref[1]) @ w2_ref[...] * _INV_SQRT_F
    acc = m
    a = attrs_ref[...]
    for v in range(D_ATTR):
        acc = acc + jnp.dot(m * a[:, v:v + 1], wskipT_ref[v],
                            preferred_element_type=jnp.float32) * _INV_SQRT_SKIP
    out_ref[...] = acc


def _final(mpart, node_attrs, W2, WskipT):
    grid = (N // _NBLK,)
    return pl.pallas_call(
        _final_body,
        grid=grid,
        in_specs=[
            pl.BlockSpec((NC, _NBLK, D_FEAT), lambda i: (0, i, 0)),
            pl.BlockSpec((_NBLK, D_ATTR), lambda i: (i, 0)),
            pl.BlockSpec((D_FEAT, D_FEAT), lambda i: (0, 0)),
            pl.BlockSpec((D_ATTR, D_FEAT, D_FEAT), lambda i: (0, 0, 0)),
        ],
        out_specs=pl.BlockSpec((_NBLK, D_FEAT), lambda i: (i, 0)),
        out_shape=jax.ShapeDtypeStruct((N, D_FEAT), jnp.float32),
    )(mpart, node_attrs, W2, WskipT)


def kernel(node_attrs, node_feats, edge_attrs, edge_feats, edge_index,
           W1, Wm1, Wm2, W2, Wskip):
    pad = E_PAD - E
    ef_pad = jnp.pad(edge_feats, ((0, pad), (0, 0)))
    ea_pad = jnp.pad(edge_attrs, ((0, pad), (0, 0)))
    sidx = jnp.pad(edge_index[0], (0, pad)).reshape(E_PAD // 128, 128)
    ridx = jnp.pad(edge_index[1], (0, pad)).reshape(E_PAD // 128, 128)
    zeros_slab = jnp.zeros((ROWS_PER_TILE, D_FEAT), jnp.float32)
    WskipT = jnp.transpose(Wskip, (1, 0, 2))

    h = _compute_h(node_feats, W1)
    tpw = _compute_tpw(ef_pad, ea_pad, Wm1, Wm2)
    mpart = _sc_scatter(h, tpw, sidx, ridx, zeros_slab)
    return _final(mpart[:, :N], node_attrs, W2, WskipT)


# speedup vs baseline: 1.6606x; 1.1393x over previous
import functools
import math

import jax
import jax.numpy as jnp
from jax import lax
from jax.experimental import pallas as pl
from jax.experimental.pallas import tpu as pltpu
from jax.experimental.pallas import tpu_sc as plsc

N = 10000
E = 320000
D_ATTR = 16
D_FEAT = 128
D_EFEAT = 8

NC = 2
NS = 16
NW = NC * NS

C = 128
T = 10240
E_PAD = T * NW
CHUNKS = T // C
N_PAD = 10240
ROWS_PER_TILE = N_PAD // NS

_INV_SQRT_F = float(1.0 / math.sqrt(D_FEAT))
_INV_SQRT_E = float(1.0 / math.sqrt(D_EFEAT))
_INV_SQRT_SKIP = float(1.0 / math.sqrt(D_FEAT * D_ATTR))
_LOG2 = float(math.log(2.0))


def _h_body(nf_ref, w1_ref, out_ref):
    out_ref[...] = jnp.dot(nf_ref[...], w1_ref[...],
                           preferred_element_type=jnp.float32) * _INV_SQRT_F


def _compute_h(node_feats, W1):
    return pl.pallas_call(
        _h_body,
        out_shape=jax.ShapeDtypeStruct((N, D_FEAT), jnp.float32),
    )(node_feats, W1)


_EBLK = 8192


def _t2_body(ef_ref, ea_ref, wm1_ref, out_ref):
    t = jax.nn.softplus(jnp.dot(ef_ref[...], wm1_ref[...],
                                preferred_element_type=jnp.float32)
                        * _INV_SQRT_E) - _LOG2
    out_ref[...] = t * ea_ref[...] * _INV_SQRT_E


def _compute_t2(ef_pad, ea_pad, Wm1):
    grid = (E_PAD // _EBLK,)
    return pl.pallas_call(
        _t2_body,
        grid=grid,
        in_specs=[
            pl.BlockSpec((_EBLK, D_EFEAT), lambda i: (i, 0)),
            pl.BlockSpec((_EBLK, 1), lambda i: (i, 0)),
            pl.BlockSpec((D_EFEAT, D_EFEAT), lambda i: (0, 0)),
        ],
        out_specs=pl.BlockSpec((_EBLK, D_EFEAT), lambda i: (i, 0)),
        out_shape=jax.ShapeDtypeStruct((E_PAD, D_EFEAT), jnp.float32),
    )(ef_pad, ea_pad, Wm1)


def _sc_body(h_hbm, t2_hbm, idx_hbm, zeros_hbm, wm2_hbm, out_hbm,
             idx0, idx1, rows0, rows1, t20, t21, wm2_v, m_shared,
             semA, semB, semI0, semI1):
    cid = lax.axis_index("c")
    sid = lax.axis_index("s")
    wid = sid * NC + cid
    cbase = wid * CHUNKS

    pltpu.sync_copy(wm2_hbm, wm2_v)
    pltpu.sync_copy(zeros_hbm, m_shared.at[pl.ds(sid * ROWS_PER_TILE,
                                                 ROWS_PER_TILE)])
    plsc.subcore_barrier()

    idx_bufs = (idx0, idx1)
    rows_bufs = (rows0, rows1)
    t2_bufs = (t20, t21)
    sems = (semA, semB)
    semsI = (semI0, semI1)

    def start_chunk(j, par):
        pltpu.async_copy(h_hbm.at[idx_bufs[par].at[0]], rows_bufs[par],
                         sems[par])
        pltpu.async_copy(t2_hbm.at[pl.ds((cbase + j) * C * D_EFEAT,
                                         C * D_EFEAT)],
                         t2_bufs[par], sems[par])

    def wait_chunk(par):
        pltpu.make_async_copy(h_hbm.at[pl.ds(0, C)], rows_bufs[par],
                              sems[par]).wait()
        pltpu.make_async_copy(t2_hbm.at[pl.ds(0, C * D_EFEAT)], t2_bufs[par],
                              sems[par]).wait()

    def start_idx(j, par):
        pltpu.async_copy(idx_hbm.at[cbase + j], idx_bufs[par], semsI[par])

    def wait_idx(par):
        pltpu.make_async_copy(idx_hbm.at[cbase], idx_bufs[par],
                              semsI[par]).wait()

    pltpu.sync_copy(idx_hbm.at[cbase], idx0)
    start_chunk(0, 0)
    start_idx(1, 1)

    def process(j, par):
        rows_v = rows_bufs[par]
        t2f = t2_bufs[par]

        for half in range(2):
            wv = [[wm2_v[k, pl.ds(half * 64 + cg * 16, 16)]
                   for k in range(D_EFEAT)] for cg in range(4)]

            def pair(p, c):
                tv = t2f[pl.ds(p * 16, 16)]
                for eo in range(2):
                    i = p * 2 + eo
                    ts = [tv[eo * D_EFEAT + k] for k in range(D_EFEAT)]
                    for cg in range(4):
                        acc = wv[cg][0] * ts[0]
                        for k in range(1, D_EFEAT):
                            acc = acc + wv[cg][k] * ts[k]
                        sl = pl.ds(half * 64 + cg * 16, 16)
                        rows_v[i, sl] = rows_v[i, sl] * acc
                return c

            lax.fori_loop(0, C // 2, pair, 0, unroll=2)

        pltpu.sync_copy(rows_v, m_shared.at[idx_bufs[par].at[1]], add=True)

    def loop(u, carry):
        for b in range(2):
            j = 2 * u + b
            nxt = 1 - b

            @pl.when(j + 1 < CHUNKS)
            def _():
                wait_idx(nxt)
                start_chunk(j + 1, nxt)

            wait_chunk(b)
            process(j, b)

            @pl.when(j + 2 < CHUNKS)
            def _():
                start_idx(j + 2, b)
        return carry

    lax.fori_loop(0, CHUNKS // 2, loop, 0, unroll=False)
    plsc.subcore_barrier()

    pltpu.sync_copy(m_shared.at[pl.ds(sid * ROWS_PER_TILE, ROWS_PER_TILE)],
                    out_hbm.at[cid, pl.ds(sid * ROWS_PER_TILE, ROWS_PER_TILE)])


def _sc_scatter(h, t2_pad, idx2, zeros_slab, Wm2):
    mesh = plsc.VectorSubcoreMesh(core_axis_name="c", subcore_axis_name="s")
    fn = functools.partial(
        pl.kernel,
        out_type=jax.ShapeDtypeStruct((NC, N_PAD, D_FEAT), jnp.float32),
        mesh=mesh,
        scratch_types=[
            pltpu.VMEM((2, 128), jnp.int32),
            pltpu.VMEM((2, 128), jnp.int32),
            pltpu.VMEM((C, D_FEAT), jnp.float32),
            pltpu.VMEM((C, D_FEAT), jnp.float32),
            pltpu.VMEM((C * D_EFEAT,), jnp.float32),
            pltpu.VMEM((C * D_EFEAT,), jnp.float32),
            pltpu.VMEM((D_EFEAT, D_FEAT), jnp.float32),
            pltpu.VMEM_SHARED((N_PAD, D_FEAT), jnp.float32),
            pltpu.SemaphoreType.DMA,
            pltpu.SemaphoreType.DMA,
            pltpu.SemaphoreType.DMA,
            pltpu.SemaphoreType.DMA,
        ],
    )(_sc_body)
    return fn(h, t2_pad, idx2, zeros_slab, Wm2)


_NBLK = 1000


def _final_body(mp_ref, attrs_ref, w2_ref, wskipT_ref, out_ref):
    m = (mp_ref[0] + mp_ref[1]) @ w2_ref[...] * _INV_SQRT_F
    acc = m
    a = attrs_ref[...]
    for v in range(D_ATTR):
        acc = acc + jnp.dot(m * a[:, v:v + 1], wskipT_ref[v],
                            preferred_element_type=jnp.float32) * _INV_SQRT_SKIP
    out_ref[...] = acc


def _final(mpart, node_attrs, W2, WskipT):
    grid = (N // _NBLK,)
    return pl.pallas_call(
        _final_body,
        grid=grid,
        in_specs=[
            pl.BlockSpec((NC, _NBLK, D_FEAT), lambda i: (0, i, 0)),
            pl.BlockSpec((_NBLK, D_ATTR), lambda i: (i, 0)),
            pl.BlockSpec((D_FEAT, D_FEAT), lambda i: (0, 0)),
            pl.BlockSpec((D_ATTR, D_FEAT, D_FEAT), lambda i: (0, 0, 0)),
        ],
        out_specs=pl.BlockSpec((_NBLK, D_FEAT), lambda i: (i, 0)),
        out_shape=jax.ShapeDtypeStruct((N, D_FEAT), jnp.float32),
    )(mpart, node_attrs, W2, WskipT)


def kernel(node_attrs, node_feats, edge_attrs, edge_feats, edge_index,
           W1, Wm1, Wm2, W2, Wskip):
    pad = E_PAD - E
    ef_pad = jnp.pad(edge_feats, ((0, pad), (0, 0)))
    ea_pad = jnp.pad(edge_attrs, ((0, pad), (0, 0)))
    sidx = jnp.pad(edge_index[0], (0, pad)).reshape(E_PAD // C, 1, C)
    ridx = jnp.pad(edge_index[1], (0, pad)).reshape(E_PAD // C, 1, C)
    idx2 = jnp.concatenate([sidx, ridx], axis=1)
    zeros_slab = jnp.zeros((ROWS_PER_TILE, D_FEAT), jnp.float32)
    WskipT = jnp.transpose(Wskip, (1, 0, 2))

    h = _compute_h(node_feats, W1)
    t2 = _compute_t2(ef_pad, ea_pad, Wm1).reshape(E_PAD * D_EFEAT)
    mpart = _sc_scatter(h, t2, idx2, zeros_slab, Wm2)
    return _final(mpart[:, :N], node_attrs, W2, WskipT)


# speedup vs baseline: 2.2573x; 1.3593x over previous
import functools
import math

import jax
import jax.numpy as jnp
from jax import lax
from jax.experimental import pallas as pl
from jax.experimental.pallas import tpu as pltpu
from jax.experimental.pallas import tpu_sc as plsc

N = 10000
E = 320000
D_ATTR = 16
D_FEAT = 128
D_EFEAT = 8

NC = 2
NS = 16
NW = NC * NS

C = 128
NCHUNK = E // C
MAIN = NCHUNK // NW
EXTRA = NCHUNK - MAIN * NW
TROW = E // 16
N_PAD = 10240
ROWS_PER_TILE = N_PAD // NS

_INV_SQRT_F = float(1.0 / math.sqrt(D_FEAT))
_INV_SQRT_E = float(1.0 / math.sqrt(D_EFEAT))
_INV_SQRT_SKIP = float(1.0 / math.sqrt(D_FEAT * D_ATTR))
_LOG2 = float(math.log(2.0))


def _h_body(nf_ref, w1_ref, out_ref):
    out_ref[...] = jnp.dot(nf_ref[...], w1_ref[...],
                           preferred_element_type=jnp.float32) * _INV_SQRT_F


def _compute_h(node_feats, W1):
    return pl.pallas_call(
        _h_body,
        out_shape=jax.ShapeDtypeStruct((N, D_FEAT), jnp.float32),
    )(node_feats, W1)


_EBLK = 2560


def _t2_body(ef_ref, ea_ref, wm1_ref, out_ref):
    t = jax.nn.softplus(jnp.dot(ef_ref[...], wm1_ref[...],
                                preferred_element_type=jnp.float32)
                        * _INV_SQRT_E) - _LOG2
    t = t * ea_ref[...] * _INV_SQRT_E
    out_ref[...] = t.T


def _compute_t2w(edge_feats, edge_attrs, Wm1):
    grid = (E // _EBLK,)
    return pl.pallas_call(
        _t2_body,
        grid=grid,
        in_specs=[
            pl.BlockSpec((_EBLK, D_EFEAT), lambda i: (i, 0)),
            pl.BlockSpec((_EBLK, 1), lambda i: (i, 0)),
            pl.BlockSpec((D_EFEAT, D_EFEAT), lambda i: (0, 0)),
        ],
        out_specs=pl.BlockSpec((D_EFEAT, _EBLK), lambda i: (0, i)),
        out_shape=jax.ShapeDtypeStruct((D_EFEAT, E), jnp.float32),
    )(edge_feats, edge_attrs, Wm1)


def _sc_body(h_hbm, t2w_hbm, eidx_hbm, ridx_hbm, zeros_hbm, wm2_hbm, out_hbm,
             sidx0, sidx1, ridx0, ridx1, rows0, rows1, t20, t21, wm2_v,
             m_shared, semA, semB, semI0, semI1):
    cid = lax.axis_index("c")
    sid = lax.axis_index("s")
    wid = sid * NC + cid
    qbase = wid * MAIN

    pltpu.sync_copy(wm2_hbm, wm2_v)
    pltpu.sync_copy(zeros_hbm, m_shared.at[pl.ds(sid * ROWS_PER_TILE,
                                                 ROWS_PER_TILE)])
    plsc.subcore_barrier()

    sidx_bufs = (sidx0, sidx1)
    ridx_bufs = (ridx0, ridx1)
    rows_bufs = (rows0, rows1)
    t2_bufs = (t20, t21)
    sems = (semA, semB)
    semsI = (semI0, semI1)

    def start_idx(q, par):
        pltpu.async_copy(eidx_hbm.at[0, pl.ds(q * C, C)], sidx_bufs[par],
                         semsI[par])
        pltpu.async_copy(ridx_hbm.at[pl.ds(q, 1)], ridx_bufs[par],
                         semsI[par])
        pltpu.async_copy(t2w_hbm.at[:, pl.ds(q * C, C)], t2_bufs[par],
                         semsI[par])

    def wait_idx(par):
        pltpu.make_async_copy(eidx_hbm.at[0, pl.ds(0, C)], sidx_bufs[par],
                              semsI[par]).wait()
        pltpu.make_async_copy(ridx_hbm.at[pl.ds(0, 1)], ridx_bufs[par],
                              semsI[par]).wait()
        pltpu.make_async_copy(t2w_hbm.at[:, pl.ds(0, C)], t2_bufs[par],
                              semsI[par]).wait()

    def start_gather(par):
        pltpu.async_copy(h_hbm.at[sidx_bufs[par]], rows_bufs[par], sems[par])

    def wait_gather(par):
        pltpu.make_async_copy(h_hbm.at[pl.ds(0, C)], rows_bufs[par],
                              sems[par]).wait()

    def process(par):
        rows_v = rows_bufs[par]
        t2_v = t2_bufs[par]

        for half in range(2):
            wv = [[wm2_v[k, pl.ds(half * 64 + cg * 16, 16)]
                   for k in range(D_EFEAT)] for cg in range(4)]

            def grp16(r, c):
                tvs = [t2_v[k, pl.ds(r * 16, 16)] for k in range(D_EFEAT)]
                for eo in range(16):
                    i = r * 16 + eo
                    ts = [tvs[k][eo] for k in range(D_EFEAT)]
                    for cg in range(4):
                        acc = wv[cg][0] * ts[0]
                        for k in range(1, D_EFEAT):
                            acc = acc + wv[cg][k] * ts[k]
                        sl = pl.ds(half * 64 + cg * 16, 16)
                        rows_v[i, sl] = rows_v[i, sl] * acc
                return c

            lax.fori_loop(0, C // 16, grp16, 0, unroll=False)

        pltpu.sync_copy(rows_v, m_shared.at[ridx_bufs[par].at[0]], add=True)

    start_idx(qbase, 0)
    wait_idx(0)
    start_gather(0)
    start_idx(qbase + 1, 1)

    def loop(u, carry):
        for b in range(2):
            j = 2 * u + b
            nxt = 1 - b

            @pl.when(j + 1 < MAIN)
            def _():
                wait_idx(nxt)
                start_gather(nxt)

            wait_gather(b)
            process(b)

            @pl.when(j + 2 < MAIN)
            def _():
                start_idx(qbase + j + 2, b)
        return carry

    lax.fori_loop(0, MAIN // 2, loop, 0, unroll=False)

    @pl.when(wid < EXTRA)
    def _():
        q = NW * MAIN + wid
        start_idx(q, 0)
        wait_idx(0)
        start_gather(0)
        wait_gather(0)
        process(0)

    plsc.subcore_barrier()

    pltpu.sync_copy(m_shared.at[pl.ds(sid * ROWS_PER_TILE, ROWS_PER_TILE)],
                    out_hbm.at[cid, pl.ds(sid * ROWS_PER_TILE, ROWS_PER_TILE)])


def _sc_scatter(h, t2w, edge_index, ridx2d, zeros_slab, Wm2):
    mesh = plsc.VectorSubcoreMesh(core_axis_name="c", subcore_axis_name="s")
    fn = functools.partial(
        pl.kernel,
        out_type=jax.ShapeDtypeStruct((NC, N_PAD, D_FEAT), jnp.float32),
        mesh=mesh,
        scratch_types=[
            pltpu.VMEM((C,), jnp.int32),
            pltpu.VMEM((C,), jnp.int32),
            pltpu.VMEM((1, C), jnp.int32),
            pltpu.VMEM((1, C), jnp.int32),
            pltpu.VMEM((C, D_FEAT), jnp.float32),
            pltpu.VMEM((C, D_FEAT), jnp.float32),
            pltpu.VMEM((8, 128), jnp.float32),
            pltpu.VMEM((8, 128), jnp.float32),
            pltpu.VMEM((D_EFEAT, D_FEAT), jnp.float32),
            pltpu.VMEM_SHARED((N_PAD, D_FEAT), jnp.float32),
            pltpu.SemaphoreType.DMA,
            pltpu.SemaphoreType.DMA,
            pltpu.SemaphoreType.DMA,
            pltpu.SemaphoreType.DMA,
        ],
    )(_sc_body)
    return fn(h, t2w, edge_index, ridx2d, zeros_slab, Wm2)


_NBLK = 1000


def _final_body(mp_ref, attrs_ref, w2_ref, wskipT_ref, out_ref):
    m = (mp_ref[0] + mp_ref[1]) @ w2_ref[...] * _INV_SQRT_F
    acc = m
    a = attrs_ref[...]
    for v in range(D_ATTR):
        acc = acc + jnp.dot(m * a[:, v:v + 1], wskipT_ref[v],
                            preferred_element_type=jnp.float32) * _INV_SQRT_SKIP
    out_ref[...] = acc


def _final(mpart, node_attrs, W2, WskipT):
    grid = (N // _NBLK,)
    return pl.pallas_call(
        _final_body,
        grid=grid,
        in_specs=[
            pl.BlockSpec((NC, _NBLK, D_FEAT), lambda i: (0, i, 0)),
            pl.BlockSpec((_NBLK, D_ATTR), lambda i: (i, 0)),
            pl.BlockSpec((D_FEAT, D_FEAT), lambda i: (0, 0)),
            pl.BlockSpec((D_ATTR, D_FEAT, D_FEAT), lambda i: (0, 0, 0)),
        ],
        out_specs=pl.BlockSpec((_NBLK, D_FEAT), lambda i: (i, 0)),
        out_shape=jax.ShapeDtypeStruct((N, D_FEAT), jnp.float32),
    )(mpart, node_attrs, W2, WskipT)


def kernel(node_attrs, node_feats, edge_attrs, edge_feats, edge_index,
           W1, Wm1, Wm2, W2, Wskip):
    ridx2d = edge_index[1].reshape(NCHUNK, C)
    zeros_slab = jnp.zeros((ROWS_PER_TILE, D_FEAT), jnp.float32)
    WskipT = jnp.transpose(Wskip, (1, 0, 2))

    h = _compute_h(node_feats, W1)
    t2w = _compute_t2w(edge_feats, edge_attrs, Wm1)
    mpart = _sc_scatter(h, t2w, edge_index, ridx2d, zeros_slab, Wm2)
    return _final(mpart, node_attrs, W2, WskipT)


# speedup vs baseline: 3.2105x; 1.4223x over previous
import functools
import math

import jax
import jax.numpy as jnp
from jax import lax
from jax.experimental import pallas as pl
from jax.experimental.pallas import tpu as pltpu
from jax.experimental.pallas import tpu_sc as plsc

N = 10000
E = 320000
D_ATTR = 16
D_FEAT = 128
D_EFEAT = 8

NC = 2
NS = 16
NW = NC * NS

C = 128
NCHUNK = E // C
MAIN = NCHUNK // NW
EXTRA = NCHUNK - MAIN * NW
TROW = E // 16
N_PAD = 10240
ROWS_PER_TILE = N_PAD // NS

_INV_SQRT_F = float(1.0 / math.sqrt(D_FEAT))
_INV_SQRT_E = float(1.0 / math.sqrt(D_EFEAT))
_INV_SQRT_SKIP = float(1.0 / math.sqrt(D_FEAT * D_ATTR))
_LOG2 = float(math.log(2.0))


def _h_body(nf_ref, w1_ref, out_ref):
    out_ref[...] = jnp.dot(nf_ref[...], w1_ref[...],
                           preferred_element_type=jnp.float32) * _INV_SQRT_F


def _compute_h(node_feats, W1):
    return pl.pallas_call(
        _h_body,
        out_shape=jax.ShapeDtypeStruct((N, D_FEAT), jnp.float32),
    )(node_feats, W1)


_EBLK = 2560


def _t2_body(efT_ref, eaT_ref, wm1_ref, out_ref):
    pre = lax.dot_general(wm1_ref[...], efT_ref[...],
                          (((0,), (0,)), ((), ())),
                          preferred_element_type=jnp.float32)
    t = jax.nn.softplus(pre * _INV_SQRT_E) - _LOG2
    out_ref[...] = t * eaT_ref[...] * _INV_SQRT_E


def _compute_t2w(efT, eaT, Wm1):
    grid = (E // _EBLK,)
    return pl.pallas_call(
        _t2_body,
        grid=grid,
        in_specs=[
            pl.BlockSpec((D_EFEAT, _EBLK), lambda i: (0, i)),
            pl.BlockSpec((1, _EBLK), lambda i: (0, i)),
            pl.BlockSpec((D_EFEAT, D_EFEAT), lambda i: (0, 0)),
        ],
        out_specs=pl.BlockSpec((D_EFEAT, _EBLK), lambda i: (0, i)),
        out_shape=jax.ShapeDtypeStruct((D_EFEAT, E), jnp.float32),
    )(efT, eaT, Wm1)


def _sc_body(h_hbm, t2w_hbm, eidx_hbm, ridx_hbm, zeros_hbm, wm2_hbm, out_hbm,
             sidx0, sidx1, ridx0, ridx1, rows0, rows1, t20, t21, wm2_v,
             m_shared, semA, semB, semI0, semI1):
    cid = lax.axis_index("c")
    sid = lax.axis_index("s")
    wid = sid * NC + cid
    qbase = wid * MAIN

    pltpu.sync_copy(wm2_hbm, wm2_v)
    pltpu.sync_copy(zeros_hbm, m_shared.at[pl.ds(sid * ROWS_PER_TILE,
                                                 ROWS_PER_TILE)])
    plsc.subcore_barrier()

    sidx_bufs = (sidx0, sidx1)
    ridx_bufs = (ridx0, ridx1)
    rows_bufs = (rows0, rows1)
    t2_bufs = (t20, t21)
    sems = (semA, semB)
    semsI = (semI0, semI1)

    def start_idx(q, par):
        pltpu.async_copy(eidx_hbm.at[0, pl.ds(q * C, C)], sidx_bufs[par],
                         semsI[par])
        pltpu.async_copy(ridx_hbm.at[pl.ds(q, 1)], ridx_bufs[par],
                         semsI[par])
        pltpu.async_copy(t2w_hbm.at[:, pl.ds(q * C, C)], t2_bufs[par],
                         semsI[par])

    def wait_idx(par):
        pltpu.make_async_copy(eidx_hbm.at[0, pl.ds(0, C)], sidx_bufs[par],
                              semsI[par]).wait()
        pltpu.make_async_copy(ridx_hbm.at[pl.ds(0, 1)], ridx_bufs[par],
                              semsI[par]).wait()
        pltpu.make_async_copy(t2w_hbm.at[:, pl.ds(0, C)], t2_bufs[par],
                              semsI[par]).wait()

    def start_gather(par):
        pltpu.async_copy(h_hbm.at[sidx_bufs[par]], rows_bufs[par], sems[par])

    def wait_gather(par):
        pltpu.make_async_copy(h_hbm.at[pl.ds(0, C)], rows_bufs[par],
                              sems[par]).wait()

    def process(par):
        rows_v = rows_bufs[par]
        t2_v = t2_bufs[par]

        for half in range(2):
            wv = [[wm2_v[k, pl.ds(half * 64 + cg * 16, 16)]
                   for k in range(D_EFEAT)] for cg in range(4)]

            def grp16(r, c):
                tvs = [t2_v[k, pl.ds(r * 16, 16)] for k in range(D_EFEAT)]
                for eo in range(16):
                    i = r * 16 + eo
                    ts = [tvs[k][eo] for k in range(D_EFEAT)]
                    for cg in range(4):
                        acc = wv[cg][0] * ts[0]
                        for k in range(1, D_EFEAT):
                            acc = acc + wv[cg][k] * ts[k]
                        sl = pl.ds(half * 64 + cg * 16, 16)
                        rows_v[i, sl] = rows_v[i, sl] * acc
                return c

            lax.fori_loop(0, C // 16, grp16, 0, unroll=False)

        pltpu.sync_copy(rows_v, m_shared.at[ridx_bufs[par].at[0]], add=True)

    start_idx(qbase, 0)
    wait_idx(0)
    start_gather(0)
    start_idx(qbase + 1, 1)

    def loop(u, carry):
        for b in range(2):
            j = 2 * u + b
            nxt = 1 - b

            @pl.when(j + 1 < MAIN)
            def _():
                wait_idx(nxt)
                start_gather(nxt)

            wait_gather(b)
            process(b)

            @pl.when(j + 2 < MAIN)
            def _():
                start_idx(qbase + j + 2, b)
        return carry

    lax.fori_loop(0, MAIN // 2, loop, 0, unroll=False)

    @pl.when(wid < EXTRA)
    def _():
        q = NW * MAIN + wid
        start_idx(q, 0)
        wait_idx(0)
        start_gather(0)
        wait_gather(0)
        process(0)

    plsc.subcore_barrier()

    pltpu.sync_copy(m_shared.at[pl.ds(sid * ROWS_PER_TILE, ROWS_PER_TILE)],
                    out_hbm.at[cid, pl.ds(sid * ROWS_PER_TILE, ROWS_PER_TILE)])


def _sc_scatter(h, t2w, edge_index, ridx2d, zeros_slab, Wm2):
    mesh = plsc.VectorSubcoreMesh(core_axis_name="c", subcore_axis_name="s")
    fn = functools.partial(
        pl.kernel,
        out_type=jax.ShapeDtypeStruct((NC, N_PAD, D_FEAT), jnp.float32),
        mesh=mesh,
        scratch_types=[
            pltpu.VMEM((C,), jnp.int32),
            pltpu.VMEM((C,), jnp.int32),
            pltpu.VMEM((1, C), jnp.int32),
            pltpu.VMEM((1, C), jnp.int32),
            pltpu.VMEM((C, D_FEAT), jnp.float32),
            pltpu.VMEM((C, D_FEAT), jnp.float32),
            pltpu.VMEM((8, 128), jnp.float32),
            pltpu.VMEM((8, 128), jnp.float32),
            pltpu.VMEM((D_EFEAT, D_FEAT), jnp.float32),
            pltpu.VMEM_SHARED((N_PAD, D_FEAT), jnp.float32),
            pltpu.SemaphoreType.DMA,
            pltpu.SemaphoreType.DMA,
            pltpu.SemaphoreType.DMA,
            pltpu.SemaphoreType.DMA,
        ],
    )(_sc_body)
    return fn(h, t2w, edge_index, ridx2d, zeros_slab, Wm2)


_NBLK = 1000


def _final_body(mp_ref, attrs_ref, w2_ref, wskipT_ref, out_ref):
    m = (mp_ref[0] + mp_ref[1]) @ w2_ref[...] * _INV_SQRT_F
    acc = m
    a = attrs_ref[...]
    for v in range(D_ATTR):
        acc = acc + jnp.dot(m * a[:, v:v + 1], wskipT_ref[v],
                            preferred_element_type=jnp.float32) * _INV_SQRT_SKIP
    out_ref[...] = acc


def _final(mpart, node_attrs, W2, WskipT):
    grid = (N // _NBLK,)
    return pl.pallas_call(
        _final_body,
        grid=grid,
        in_specs=[
            pl.BlockSpec((NC, _NBLK, D_FEAT), lambda i: (0, i, 0)),
            pl.BlockSpec((_NBLK, D_ATTR), lambda i: (i, 0)),
            pl.BlockSpec((D_FEAT, D_FEAT), lambda i: (0, 0)),
            pl.BlockSpec((D_ATTR, D_FEAT, D_FEAT), lambda i: (0, 0, 0)),
        ],
        out_specs=pl.BlockSpec((_NBLK, D_FEAT), lambda i: (i, 0)),
        out_shape=jax.ShapeDtypeStruct((N, D_FEAT), jnp.float32),
    )(mpart, node_attrs, W2, WskipT)


def kernel(node_attrs, node_feats, edge_attrs, edge_feats, edge_index,
           W1, Wm1, Wm2, W2, Wskip):
    ridx2d = edge_index[1].reshape(NCHUNK, C)
    zeros_slab = jnp.zeros((ROWS_PER_TILE, D_FEAT), jnp.float32)
    WskipT = jnp.transpose(Wskip, (1, 0, 2))

    h = _compute_h(node_feats, W1)
    t2w = _compute_t2w(edge_feats.T, edge_attrs.T, Wm1)
    mpart = _sc_scatter(h, t2w, edge_index, ridx2d, zeros_slab, Wm2)
    return _final(mpart, node_attrs, W2, WskipT)


# speedup vs baseline: 3.4650x; 1.0792x over previous
import functools
import math

import jax
import jax.numpy as jnp
from jax import lax
from jax.experimental import pallas as pl
from jax.experimental.pallas import tpu as pltpu
from jax.experimental.pallas import tpu_sc as plsc

N = 10000
E = 320000
D_ATTR = 16
D_FEAT = 128
D_EFEAT = 8

NC = 2
NS = 16
NW = NC * NS

C = 128
NCHUNK = E // C
MAIN = NCHUNK // NW
EXTRA = NCHUNK - MAIN * NW
TROW = E // 16
N_PAD = 10240
ROWS_PER_TILE = N_PAD // NS

_INV_SQRT_F = float(1.0 / math.sqrt(D_FEAT))
_INV_SQRT_E = float(1.0 / math.sqrt(D_EFEAT))
_INV_SQRT_SKIP = float(1.0 / math.sqrt(D_FEAT * D_ATTR))
_LOG2 = float(math.log(2.0))


def _h_body(nf_ref, w1_ref, out_ref):
    out_ref[...] = jnp.dot(nf_ref[...], w1_ref[...],
                           preferred_element_type=jnp.float32) * _INV_SQRT_F


def _compute_h(node_feats, W1):
    return pl.pallas_call(
        _h_body,
        out_shape=jax.ShapeDtypeStruct((N, D_FEAT), jnp.float32),
    )(node_feats, W1)


_EBLK = 2560


def _t2_body(efT_ref, eaT_ref, wm1_ref, out_ref):
    pre = lax.dot_general(wm1_ref[...], efT_ref[...],
                          (((0,), (0,)), ((), ())),
                          preferred_element_type=jnp.float32)
    t = jax.nn.softplus(pre * _INV_SQRT_E) - _LOG2
    out_ref[...] = t * eaT_ref[...] * _INV_SQRT_E


def _compute_t2w(efT, eaT, Wm1):
    grid = (E // _EBLK,)
    return pl.pallas_call(
        _t2_body,
        grid=grid,
        in_specs=[
            pl.BlockSpec((D_EFEAT, _EBLK), lambda i: (0, i)),
            pl.BlockSpec((1, _EBLK), lambda i: (0, i)),
            pl.BlockSpec((D_EFEAT, D_EFEAT), lambda i: (0, 0)),
        ],
        out_specs=pl.BlockSpec((D_EFEAT, _EBLK), lambda i: (0, i)),
        out_shape=jax.ShapeDtypeStruct((D_EFEAT, E), jnp.float32),
    )(efT, eaT, Wm1)


def _sc_body(h_hbm, t2w_hbm, eidx_hbm, ridx_hbm, zeros_hbm, wm2_hbm, out_hbm,
             sidx0, sidx1, ridx0, ridx1, srdx0, srdx1, rows0, rows1,
             t20, t21, wm2_v, m_shared,
             semA, semB, semI0, semI1, semS0, semS1):
    cid = lax.axis_index("c")
    sid = lax.axis_index("s")
    wid = sid * NC + cid
    qbase = wid * MAIN

    pltpu.sync_copy(wm2_hbm, wm2_v)
    pltpu.sync_copy(zeros_hbm, m_shared.at[pl.ds(sid * ROWS_PER_TILE,
                                                 ROWS_PER_TILE)])
    plsc.subcore_barrier()

    sidx_bufs = (sidx0, sidx1)
    ridx_bufs = (ridx0, ridx1)
    srdx_bufs = (srdx0, srdx1)
    rows_bufs = (rows0, rows1)
    t2_bufs = (t20, t21)
    sems = (semA, semB)
    semsI = (semI0, semI1)
    semsS = (semS0, semS1)

    def start_idx(q, par):
        pltpu.async_copy(eidx_hbm.at[0, pl.ds(q * C, C)], sidx_bufs[par],
                         semsI[par])
        pltpu.async_copy(ridx_hbm.at[pl.ds(q, 1)], ridx_bufs[par],
                         semsI[par])
        pltpu.async_copy(t2w_hbm.at[:, pl.ds(q * C, C)], t2_bufs[par],
                         semsI[par])

    def wait_idx(par):
        pltpu.make_async_copy(eidx_hbm.at[0, pl.ds(0, C)], sidx_bufs[par],
                              semsI[par]).wait()
        pltpu.make_async_copy(ridx_hbm.at[pl.ds(0, 1)], ridx_bufs[par],
                              semsI[par]).wait()
        pltpu.make_async_copy(t2w_hbm.at[:, pl.ds(0, C)], t2_bufs[par],
                              semsI[par]).wait()

    def start_gather(par):
        pltpu.async_copy(h_hbm.at[sidx_bufs[par]], rows_bufs[par], sems[par])

    def wait_gather(par):
        pltpu.make_async_copy(h_hbm.at[pl.ds(0, C)], rows_bufs[par],
                              sems[par]).wait()

    def start_scatter(par):
        for g in range(8):
            sl = pl.ds(g * 16, 16)
            srdx_bufs[par][0, sl] = ridx_bufs[par][0, sl]
        pltpu.async_copy(rows_bufs[par], m_shared.at[srdx_bufs[par].at[0]],
                         semsS[par], add=True)

    def wait_scatter(par):
        pltpu.make_async_copy(h_hbm.at[pl.ds(0, C)], rows_bufs[par],
                              semsS[par]).wait()

    def process(par):
        rows_v = rows_bufs[par]
        t2_v = t2_bufs[par]

        for half in range(2):
            wv = [[wm2_v[k, pl.ds(half * 64 + cg * 16, 16)]
                   for k in range(D_EFEAT)] for cg in range(4)]

            def grp16(r, c):
                tvs = [t2_v[k, pl.ds(r * 16, 16)] for k in range(D_EFEAT)]
                for eo in range(16):
                    i = r * 16 + eo
                    ts = [tvs[k][eo] for k in range(D_EFEAT)]
                    for cg in range(4):
                        acc = wv[cg][0] * ts[0]
                        for k in range(1, D_EFEAT):
                            acc = acc + wv[cg][k] * ts[k]
                        sl = pl.ds(half * 64 + cg * 16, 16)
                        rows_v[i, sl] = rows_v[i, sl] * acc
                return c

            lax.fori_loop(0, C // 16, grp16, 0, unroll=False)

    start_idx(qbase, 0)
    wait_idx(0)
    start_gather(0)
    start_idx(qbase + 1, 1)

    def loop(u, carry):
        for b in range(2):
            j = 2 * u + b
            nxt = 1 - b

            @pl.when(j + 1 < MAIN)
            def _():
                @pl.when(j >= 1)
                def _():
                    wait_scatter(nxt)
                wait_idx(nxt)
                start_gather(nxt)

            wait_gather(b)
            process(b)
            start_scatter(b)

            @pl.when(j + 2 < MAIN)
            def _():
                start_idx(qbase + j + 2, b)
        return carry

    lax.fori_loop(0, MAIN // 2, loop, 0, unroll=False)
    wait_scatter(0)
    wait_scatter(1)

    @pl.when(wid < EXTRA)
    def _():
        q = NW * MAIN + wid
        start_idx(q, 0)
        wait_idx(0)
        start_gather(0)
        wait_gather(0)
        process(0)
        pltpu.sync_copy(rows0, m_shared.at[ridx0.at[0]], add=True)

    plsc.subcore_barrier()

    pltpu.sync_copy(m_shared.at[pl.ds(sid * ROWS_PER_TILE, ROWS_PER_TILE)],
                    out_hbm.at[cid, pl.ds(sid * ROWS_PER_TILE, ROWS_PER_TILE)])


def _sc_scatter(h, t2w, edge_index, ridx2d, zeros_slab, Wm2):
    mesh = plsc.VectorSubcoreMesh(core_axis_name="c", subcore_axis_name="s")
    fn = functools.partial(
        pl.kernel,
        out_type=jax.ShapeDtypeStruct((NC, N_PAD, D_FEAT), jnp.float32),
        mesh=mesh,
        scratch_types=[
            pltpu.VMEM((C,), jnp.int32),
            pltpu.VMEM((C,), jnp.int32),
            pltpu.VMEM((1, C), jnp.int32),
            pltpu.VMEM((1, C), jnp.int32),
            pltpu.VMEM((1, C), jnp.int32),
            pltpu.VMEM((1, C), jnp.int32),
            pltpu.VMEM((C, D_FEAT), jnp.float32),
            pltpu.VMEM((C, D_FEAT), jnp.float32),
            pltpu.VMEM((8, 128), jnp.float32),
            pltpu.VMEM((8, 128), jnp.float32),
            pltpu.VMEM((D_EFEAT, D_FEAT), jnp.float32),
            pltpu.VMEM_SHARED((N_PAD, D_FEAT), jnp.float32),
            pltpu.SemaphoreType.DMA,
            pltpu.SemaphoreType.DMA,
            pltpu.SemaphoreType.DMA,
            pltpu.SemaphoreType.DMA,
            pltpu.SemaphoreType.DMA,
            pltpu.SemaphoreType.DMA,
        ],
    )(_sc_body)
    return fn(h, t2w, edge_index, ridx2d, zeros_slab, Wm2)


_NBLK = 1000


def _final_body(mp_ref, attrs_ref, w2_ref, wskipT_ref, out_ref):
    m = (mp_ref[0] + mp_ref[1]) @ w2_ref[...] * _INV_SQRT_F
    acc = m
    a = attrs_ref[...]
    for v in range(D_ATTR):
        acc = acc + jnp.dot(m * a[:, v:v + 1], wskipT_ref[v],
                            preferred_element_type=jnp.float32) * _INV_SQRT_SKIP
    out_ref[...] = acc


def _final(mpart, node_attrs, W2, WskipT):
    grid = (N // _NBLK,)
    return pl.pallas_call(
        _final_body,
        grid=grid,
        in_specs=[
            pl.BlockSpec((NC, _NBLK, D_FEAT), lambda i: (0, i, 0)),
            pl.BlockSpec((_NBLK, D_ATTR), lambda i: (i, 0)),
            pl.BlockSpec((D_FEAT, D_FEAT), lambda i: (0, 0)),
            pl.BlockSpec((D_ATTR, D_FEAT, D_FEAT), lambda i: (0, 0, 0)),
        ],
        out_specs=pl.BlockSpec((_NBLK, D_FEAT), lambda i: (i, 0)),
        out_shape=jax.ShapeDtypeStruct((N, D_FEAT), jnp.float32),
    )(mpart, node_attrs, W2, WskipT)


def kernel(node_attrs, node_feats, edge_attrs, edge_feats, edge_index,
           W1, Wm1, Wm2, W2, Wskip):
    ridx2d = edge_index[1].reshape(NCHUNK, C)
    zeros_slab = jnp.zeros((ROWS_PER_TILE, D_FEAT), jnp.float32)
    WskipT = jnp.transpose(Wskip, (1, 0, 2))

    h = _compute_h(node_feats, W1)
    t2w = _compute_t2w(edge_feats.T, edge_attrs.T, Wm1)
    mpart = _sc_scatter(h, t2w, edge_index, ridx2d, zeros_slab, Wm2)
    return _final(mpart, node_attrs, W2, WskipT)
